# baseline (speedup 1.0000x reference)
import functools

import jax
import jax.numpy as jnp
from jax import lax
from jax.experimental import pallas as pl
from jax.experimental.pallas import tpu as pltpu
from jax.experimental.pallas import tpu_sc as plsc

N = 10000
D = 128
EPS = 1e-5

N_PAD = 10112
ROWS_PER_TILE = 632
E_PAD = 327680
EDGES_PER_TILE = 10240
CHUNK = 128
NCHUNKS = EDGES_PER_TILE // CHUNK
DEG_W = 16


def _sc_mesh():
    return plsc.VectorSubcoreMesh(core_axis_name="c", subcore_axis_name="s")


def _edge_scatter_call(table, src_p, dst_p, zeros, width):
    @functools.partial(
        pl.kernel,
        mesh=_sc_mesh(),
        out_type=jax.ShapeDtypeStruct((2 * N_PAD, width), jnp.float32),
        scratch_types=[
            pltpu.VMEM((CHUNK,), jnp.int32),
            pltpu.VMEM((CHUNK,), jnp.int32),
            pltpu.VMEM((CHUNK, width), jnp.float32),
            pltpu.VMEM_SHARED((N_PAD, width), jnp.float32),
            pltpu.SemaphoreType.DMA,
        ],
    )
    def k(table_hbm, src_hbm, dst_hbm, zeros_hbm, out_hbm,
          sidx, didx, rows, agg_sh, sem):
        c = lax.axis_index("c")
        s = lax.axis_index("s")
        wid = s * 2 + c
        row0 = s * ROWS_PER_TILE
        pltpu.sync_copy(zeros_hbm.at[pl.ds(row0, ROWS_PER_TILE)],
                        agg_sh.at[pl.ds(row0, ROWS_PER_TILE)])
        plsc.subcore_barrier()
        ebase = wid * EDGES_PER_TILE

        def body(i, carry):
            e0 = ebase + i * CHUNK
            pltpu.sync_copy(src_hbm.at[pl.ds(e0, CHUNK)], sidx)
            pltpu.sync_copy(dst_hbm.at[pl.ds(e0, CHUNK)], didx)
            pltpu.async_copy(table_hbm.at[sidx], rows, sem).wait()
            pltpu.sync_copy(rows, agg_sh.at[didx], add=True)
            return carry

        lax.fori_loop(0, NCHUNKS, body, 0)
        plsc.subcore_barrier()
        pltpu.sync_copy(agg_sh.at[pl.ds(row0, ROWS_PER_TILE)],
                        out_hbm.at[pl.ds(c * N_PAD + row0, ROWS_PER_TILE)])

    return k(table, src_p, dst_p, zeros)


def _proj_bn_call(x, W_proj, b_proj, gamma, beta, degp):
    def body(x_ref, wp_ref, bp_ref, g_ref, bt_ref, degp_ref, hs0_ref, isd_ref):
        h = jnp.dot(x_ref[...], wp_ref[...],
                    preferred_element_type=jnp.float32) + bp_ref[...]
        mean = jnp.mean(h, axis=0, keepdims=True)
        ctr = h - mean
        var = jnp.mean(ctr * ctr, axis=0, keepdims=True)
        hbn = ctr * lax.rsqrt(var + EPS) * g_ref[...] + bt_ref[...]
        deg = (degp_ref[0:N_PAD, 0:1] + degp_ref[N_PAD:2 * N_PAD, 0:1]) + 1.0
        isd = lax.rsqrt(deg)
        rowid = lax.broadcasted_iota(jnp.int32, (N_PAD, 1), 0)
        isd = jnp.where(rowid < N, isd, 0.0)
        isd_b = jnp.broadcast_to(isd, (N_PAD, D))
        isd_ref[...] = isd_b
        hs0_ref[0:N, :] = isd_b[0:N, :] * hbn
        hs0_ref[N:N_PAD, :] = jnp.zeros((N_PAD - N, D), jnp.float32)

    return pl.pallas_call(
        body,
        out_shape=(jax.ShapeDtypeStruct((N_PAD, D), jnp.float32),
                   jax.ShapeDtypeStruct((N_PAD, D), jnp.float32)),
    )(x, W_proj, b_proj, gamma, beta, degp)


def _layer_call(aggp, hs_prev, isd_b, W, b):
    def body(aggp_ref, hs_ref, isd_ref, w_ref, b_ref, out_ref):
        isd = isd_ref[...]
        full = isd * (aggp_ref[0:N_PAD, :] + aggp_ref[N_PAD:2 * N_PAD, :]
                      + hs_ref[...])
        h = jnp.maximum(jnp.dot(full, w_ref[...],
                                preferred_element_type=jnp.float32)
                        + b_ref[...], 0.0)
        out_ref[...] = isd * h

    return pl.pallas_call(
        body,
        out_shape=jax.ShapeDtypeStruct((N_PAD, D), jnp.float32),
    )(aggp, hs_prev, isd_b, W, b)


def _final_call(aggp, hs_prev, isd_b, W1, b1, W_out, b_out):
    def body(aggp_ref, hs_ref, isd_ref, w1_ref, b1_ref, wo_ref, bo_ref,
             out_ref):
        isd = isd_ref[...]
        full = isd * (aggp_ref[0:N_PAD, :] + aggp_ref[N_PAD:2 * N_PAD, :]
                      + hs_ref[...])
        h = jnp.maximum(jnp.dot(full, w1_ref[...],
                                preferred_element_type=jnp.float32)
                        + b1_ref[...], 0.0)
        out = jnp.dot(h[0:N, :], wo_ref[...],
                      preferred_element_type=jnp.float32) + bo_ref[...]
        out_ref[...] = out

    return pl.pallas_call(
        body,
        out_shape=jax.ShapeDtypeStruct((N, 3), jnp.float32),
    )(aggp, hs_prev, isd_b, W1, b1, W_out, b_out)


def kernel(x, edge_index, W_proj, b_proj, gamma, beta, W0, b0, W1, b1,
           W_out, b_out):
    e = edge_index.shape[1]
    pad = jnp.full((E_PAD - e,), N, dtype=edge_index.dtype)
    src_p = jnp.concatenate([edge_index[0], pad])
    dst_p = jnp.concatenate([edge_index[1], pad])
    zeros = jnp.zeros((N_PAD, D), jnp.float32)
    ones_table = jnp.ones((N_PAD, D), jnp.float32)
    gidx = jnp.zeros((E_PAD,), dtype=edge_index.dtype)

    degp = _edge_scatter_call(ones_table, gidx, dst_p, zeros, D)
    hs0, isd_b = _proj_bn_call(x, W_proj, b_proj.reshape(1, D),
                               gamma.reshape(1, D), beta.reshape(1, D), degp)
    agg1 = _edge_scatter_call(hs0, src_p, dst_p, zeros, D)
    hs1 = _layer_call(agg1, hs0, isd_b, W0, b0.reshape(1, D))
    agg2 = _edge_scatter_call(hs1, src_p, dst_p, zeros, D)
    out = _final_call(agg2, hs1, isd_b, W1, b1.reshape(1, D),
                      W_out, b_out.reshape(1, 3))
    return out


# speedup vs baseline: 7.0276x; 7.0276x over previous
import functools

import jax
import jax.numpy as jnp
from jax import lax
from jax.experimental import pallas as pl
from jax.experimental.pallas import tpu as pltpu
from jax.experimental.pallas import tpu_sc as plsc

N = 10000
D = 128
EPS = 1e-5

N_PAD = 10112
ROWS_PER_TILE = 632
E_PAD = 327680
EDGES_PER_TILE = 10240
CHUNK = 128
NCHUNKS = EDGES_PER_TILE // CHUNK
DEG_W = 16


def _sc_mesh():
    return plsc.VectorSubcoreMesh(core_axis_name="c", subcore_axis_name="s")


def _edge_scatter_call(table, src_p, dst_p, zeros, width):
    @functools.partial(
        pl.kernel,
        mesh=_sc_mesh(),
        out_type=jax.ShapeDtypeStruct((2 * N_PAD, width), jnp.float32),
        scratch_types=[
            pltpu.VMEM((CHUNK,), jnp.int32),
            pltpu.VMEM((CHUNK,), jnp.int32),
            pltpu.VMEM((CHUNK, width), jnp.float32),
            pltpu.VMEM_SHARED((N_PAD, width), jnp.float32),
            pltpu.SemaphoreType.DMA,
        ],
    )
    def k(table_hbm, src_hbm, dst_hbm, zeros_hbm, out_hbm,
          sidx, didx, rows, agg_sh, sem):
        c = lax.axis_index("c")
        s = lax.axis_index("s")
        wid = s * 2 + c
        row0 = s * ROWS_PER_TILE
        pltpu.sync_copy(zeros_hbm.at[pl.ds(row0, ROWS_PER_TILE)],
                        agg_sh.at[pl.ds(row0, ROWS_PER_TILE)])
        plsc.subcore_barrier()
        ebase = wid * EDGES_PER_TILE

        def body(i, carry):
            e0 = ebase + i * CHUNK
            pltpu.sync_copy(src_hbm.at[pl.ds(e0, CHUNK)], sidx)
            pltpu.sync_copy(dst_hbm.at[pl.ds(e0, CHUNK)], didx)
            pltpu.async_copy(table_hbm.at[sidx], rows, sem).wait()
            pltpu.sync_copy(rows, agg_sh.at[didx], add=True)
            return carry

        lax.fori_loop(0, NCHUNKS, body, 0)
        plsc.subcore_barrier()
        pltpu.sync_copy(agg_sh.at[pl.ds(row0, ROWS_PER_TILE)],
                        out_hbm.at[pl.ds(c * N_PAD + row0, ROWS_PER_TILE)])

    return k(table, src_p, dst_p, zeros)


def _proj_bn_call(x, W_proj, b_proj, gamma, beta, degp):
    def body(x_ref, wp_ref, bp_ref, g_ref, bt_ref, degp_ref, hs0_ref, isd_ref):
        h = jnp.dot(x_ref[...], wp_ref[...],
                    preferred_element_type=jnp.float32) + bp_ref[...]
        mean = jnp.mean(h, axis=0, keepdims=True)
        ctr = h - mean
        var = jnp.mean(ctr * ctr, axis=0, keepdims=True)
        hbn = ctr * lax.rsqrt(var + EPS) * g_ref[...] + bt_ref[...]
        deg = (degp_ref[0:N_PAD, 0:1] + degp_ref[N_PAD:2 * N_PAD, 0:1]) + 1.0
        isd = lax.rsqrt(deg)
        rowid = lax.broadcasted_iota(jnp.int32, (N_PAD, 1), 0)
        isd = jnp.where(rowid < N, isd, 0.0)
        isd_b = jnp.broadcast_to(isd, (N_PAD, D))
        isd_ref[...] = isd_b
        hs0_ref[0:N, :] = isd_b[0:N, :] * hbn
        hs0_ref[N:N_PAD, :] = jnp.zeros((N_PAD - N, D), jnp.float32)

    return pl.pallas_call(
        body,
        out_shape=(jax.ShapeDtypeStruct((N_PAD, D), jnp.float32),
                   jax.ShapeDtypeStruct((N_PAD, D), jnp.float32)),
    )(x, W_proj, b_proj, gamma, beta, degp)


def _layer_call(aggp, hs_prev, isd_b, W, b):
    def body(aggp_ref, hs_ref, isd_ref, w_ref, b_ref, out_ref):
        isd = isd_ref[...]
        full = isd * (aggp_ref[0:N_PAD, :] + aggp_ref[N_PAD:2 * N_PAD, :]
                      + hs_ref[...])
        h = jnp.maximum(jnp.dot(full, w_ref[...],
                                preferred_element_type=jnp.float32)
                        + b_ref[...], 0.0)
        out_ref[...] = isd * h

    return pl.pallas_call(
        body,
        out_shape=jax.ShapeDtypeStruct((N_PAD, D), jnp.float32),
    )(aggp, hs_prev, isd_b, W, b)


def _final_call(aggp, hs_prev, isd_b, W1, b1, W_out, b_out):
    def body(aggp_ref, hs_ref, isd_ref, w1_ref, b1_ref, wo_ref, bo_ref,
             out_ref):
        isd = isd_ref[...]
        full = isd * (aggp_ref[0:N_PAD, :] + aggp_ref[N_PAD:2 * N_PAD, :]
                      + hs_ref[...])
        h = jnp.maximum(jnp.dot(full, w1_ref[...],
                                preferred_element_type=jnp.float32)
                        + b1_ref[...], 0.0)
        out = jnp.dot(h[0:N, :], wo_ref[...],
                      preferred_element_type=jnp.float32) + bo_ref[...]
        out_ref[...] = out

    return pl.pallas_call(
        body,
        out_shape=jax.ShapeDtypeStruct((N, 3), jnp.float32),
    )(aggp, hs_prev, isd_b, W1, b1, W_out, b_out)


def kernel(x, edge_index, W_proj, b_proj, gamma, beta, W0, b0, W1, b1,
           W_out, b_out):
    e = edge_index.shape[1]
    pad = jnp.full((E_PAD - e,), N, dtype=edge_index.dtype)
    src_p = jnp.concatenate([edge_index[0], pad])
    dst_p = jnp.concatenate([edge_index[1], pad])
    zeros = jnp.zeros((N_PAD, D), jnp.float32)
    ones_table = jnp.ones((N_PAD, D), jnp.float32)

    degp = _edge_scatter_call(ones_table, dst_p, dst_p, zeros, D)
    hs0, isd_b = _proj_bn_call(x, W_proj, b_proj.reshape(1, D),
                               gamma.reshape(1, D), beta.reshape(1, D), degp)
    agg1 = _edge_scatter_call(hs0, src_p, dst_p, zeros, D)
    hs1 = _layer_call(agg1, hs0, isd_b, W0, b0.reshape(1, D))
    agg2 = _edge_scatter_call(hs1, src_p, dst_p, zeros, D)
    out = _final_call(agg2, hs1, isd_b, W1, b1.reshape(1, D),
                      W_out, b_out.reshape(1, 3))
    return out


# speedup vs baseline: 7.5758x; 1.0780x over previous
import functools

import jax
import jax.numpy as jnp
from jax import lax
from jax.experimental import pallas as pl
from jax.experimental.pallas import tpu as pltpu
from jax.experimental.pallas import tpu_sc as plsc

N = 10000
D = 128
EPS = 1e-5

N_PAD = 10112
ROWS_PER_TILE = 632
E_PAD = 327680
EDGES_PER_TILE = 10240
CHUNK = 128
NCHUNKS = EDGES_PER_TILE // CHUNK
DEG_W = 16


def _sc_mesh():
    return plsc.VectorSubcoreMesh(core_axis_name="c", subcore_axis_name="s")


def _edge_scatter_call(table, src_p, dst_p, zeros, width):
    @functools.partial(
        pl.kernel,
        mesh=_sc_mesh(),
        out_type=jax.ShapeDtypeStruct((2 * N_PAD, width), jnp.float32),
        scratch_types=[
            pltpu.VMEM((CHUNK,), jnp.int32),
            pltpu.VMEM((CHUNK,), jnp.int32),
            pltpu.VMEM((CHUNK,), jnp.int32),
            pltpu.VMEM((CHUNK,), jnp.int32),
            pltpu.VMEM((CHUNK, width), jnp.float32),
            pltpu.VMEM((CHUNK, width), jnp.float32),
            pltpu.SemaphoreType.DMA,
            pltpu.VMEM_SHARED((N_PAD, width), jnp.float32),
        ],
    )
    def k(table_hbm, src_hbm, dst_hbm, zeros_hbm, out_hbm,
          sidx0, didx0, sidx1, didx1, rows0, rows1, sem, agg_sh):
        c = lax.axis_index("c")
        s = lax.axis_index("s")
        wid = s * 2 + c
        row0 = s * ROWS_PER_TILE
        pltpu.sync_copy(zeros_hbm.at[pl.ds(row0, ROWS_PER_TILE)],
                        agg_sh.at[pl.ds(row0, ROWS_PER_TILE)])
        plsc.subcore_barrier()
        ebase = wid * EDGES_PER_TILE
        sidx = (sidx0, sidx1)
        didx = (didx0, didx1)
        rows = (rows0, rows1)

        pltpu.sync_copy(src_hbm.at[pl.ds(ebase, CHUNK)], sidx0)
        pltpu.sync_copy(dst_hbm.at[pl.ds(ebase, CHUNK)], didx0)
        pltpu.async_copy(table_hbm.at[sidx0], rows0, sem).wait()

        def body2(i, b, nb):
            e0 = ebase + (i + 1) * CHUNK
            pltpu.sync_copy(src_hbm.at[pl.ds(e0, CHUNK)], sidx[nb])
            pltpu.sync_copy(dst_hbm.at[pl.ds(e0, CHUNK)], didx[nb])
            cp = pltpu.async_copy(table_hbm.at[sidx[nb]], rows[nb], sem)
            pltpu.sync_copy(rows[b], agg_sh.at[didx[b]], add=True)
            cp.wait()

        def body(i2, carry):
            i = i2 * 2
            body2(i, 0, 1)
            body2(i + 1, 1, 0)
            return carry

        lax.fori_loop(0, (NCHUNKS - 2) // 2, body, 0)
        body2(NCHUNKS - 2, 0, 1)
        pltpu.sync_copy(rows1, agg_sh.at[didx1], add=True)

        plsc.subcore_barrier()
        pltpu.sync_copy(agg_sh.at[pl.ds(row0, ROWS_PER_TILE)],
                        out_hbm.at[pl.ds(c * N_PAD + row0, ROWS_PER_TILE)])

    return k(table, src_p, dst_p, zeros)


def _proj_bn_call(x, W_proj, b_proj, gamma, beta, degp):
    def body(x_ref, wp_ref, bp_ref, g_ref, bt_ref, degp_ref, hs0_ref, isd_ref):
        h = jnp.dot(x_ref[...], wp_ref[...],
                    preferred_element_type=jnp.float32) + bp_ref[...]
        mean = jnp.mean(h, axis=0, keepdims=True)
        ctr = h - mean
        var = jnp.mean(ctr * ctr, axis=0, keepdims=True)
        hbn = ctr * lax.rsqrt(var + EPS) * g_ref[...] + bt_ref[...]
        deg = (degp_ref[0:N_PAD, 0:1] + degp_ref[N_PAD:2 * N_PAD, 0:1]) + 1.0
        isd = lax.rsqrt(deg)
        rowid = lax.broadcasted_iota(jnp.int32, (N_PAD, 1), 0)
        isd = jnp.where(rowid < N, isd, 0.0)
        isd_b = jnp.broadcast_to(isd, (N_PAD, D))
        isd_ref[...] = isd_b
        hs0_ref[0:N, :] = isd_b[0:N, :] * hbn
        hs0_ref[N:N_PAD, :] = jnp.zeros((N_PAD - N, D), jnp.float32)

    return pl.pallas_call(
        body,
        out_shape=(jax.ShapeDtypeStruct((N_PAD, D), jnp.float32),
                   jax.ShapeDtypeStruct((N_PAD, D), jnp.float32)),
    )(x, W_proj, b_proj, gamma, beta, degp)


def _layer_call(aggp, hs_prev, isd_b, W, b):
    def body(aggp_ref, hs_ref, isd_ref, w_ref, b_ref, out_ref):
        isd = isd_ref[...]
        full = isd * (aggp_ref[0:N_PAD, :] + aggp_ref[N_PAD:2 * N_PAD, :]
                      + hs_ref[...])
        h = jnp.maximum(jnp.dot(full, w_ref[...],
                                preferred_element_type=jnp.float32)
                        + b_ref[...], 0.0)
        out_ref[...] = isd * h

    return pl.pallas_call(
        body,
        out_shape=jax.ShapeDtypeStruct((N_PAD, D), jnp.float32),
    )(aggp, hs_prev, isd_b, W, b)


def _final_call(aggp, hs_prev, isd_b, W1, b1, W_out, b_out):
    def body(aggp_ref, hs_ref, isd_ref, w1_ref, b1_ref, wo_ref, bo_ref,
             out_ref):
        isd = isd_ref[...]
        full = isd * (aggp_ref[0:N_PAD, :] + aggp_ref[N_PAD:2 * N_PAD, :]
                      + hs_ref[...])
        h = jnp.maximum(jnp.dot(full, w1_ref[...],
                                preferred_element_type=jnp.float32)
                        + b1_ref[...], 0.0)
        out = jnp.dot(h[0:N, :], wo_ref[...],
                      preferred_element_type=jnp.float32) + bo_ref[...]
        out_ref[...] = out

    return pl.pallas_call(
        body,
        out_shape=jax.ShapeDtypeStruct((N, 3), jnp.float32),
    )(aggp, hs_prev, isd_b, W1, b1, W_out, b_out)


def kernel(x, edge_index, W_proj, b_proj, gamma, beta, W0, b0, W1, b1,
           W_out, b_out):
    e = edge_index.shape[1]
    pad = jnp.full((E_PAD - e,), N, dtype=edge_index.dtype)
    src_p = jnp.concatenate([edge_index[0], pad])
    dst_p = jnp.concatenate([edge_index[1], pad])
    zeros = jnp.zeros((N_PAD, D), jnp.float32)
    ones_table = jnp.ones((N_PAD, D), jnp.float32)

    degp = _edge_scatter_call(ones_table, dst_p, dst_p, zeros, D)
    hs0, isd_b = _proj_bn_call(x, W_proj, b_proj.reshape(1, D),
                               gamma.reshape(1, D), beta.reshape(1, D), degp)
    agg1 = _edge_scatter_call(hs0, src_p, dst_p, zeros, D)
    hs1 = _layer_call(agg1, hs0, isd_b, W0, b0.reshape(1, D))
    agg2 = _edge_scatter_call(hs1, src_p, dst_p, zeros, D)
    out = _final_call(agg2, hs1, isd_b, W1, b1.reshape(1, D),
                      W_out, b_out.reshape(1, 3))
    return out
